# baseline (speedup 1.0000x reference)
import jax
import jax.numpy as jnp
from jax.experimental import pallas as pl
from jax.experimental.pallas import tpu as pltpu

_NUM_BOXES = 100
_LEVELS = ((8, 200, 336), (8, 100, 168), (8, 50, 84))


def _lamm_body(h0_ref, h1_ref, h2_ref, lab_ref, dx_ref, dy_ref, out_ref,
               v0_ref, v1_ref, v2_ref, sem0, sem1, sem2):
    cp0 = pltpu.make_async_copy(h0_ref, v0_ref, sem0)
    cp1 = pltpu.make_async_copy(h1_ref, v1_ref, sem1)
    cp2 = pltpu.make_async_copy(h2_ref, v2_ref, sem2)
    cp2.start()
    cp1.start()
    cp0.start()

    dimx = dx_ref[0, 0]
    dimy = dy_ref[0, 0]
    lab = lab_ref[:, :]

    areas = []
    for _, hgt, wid in _LEVELS:
        sx = wid / dimx
        sy = hgt / dimy
        x1 = jnp.clip(jnp.round(lab[:, 0:1] * sx), 0.0, wid - 1.0)
        y1 = jnp.clip(jnp.round(lab[:, 1:2] * sy), 0.0, hgt - 1.0)
        x2 = jnp.clip(jnp.round(lab[:, 2:3] * sx), 0.0, float(wid))
        y2 = jnp.clip(jnp.round(lab[:, 3:4] * sy), 0.0, float(hgt))
        valid = ((x2 > x1) & (y2 > y1)).astype(jnp.float32)
        xx = jax.lax.broadcasted_iota(
            jnp.int32, (_NUM_BOXES, wid), 1).astype(jnp.float32)
        yy = jax.lax.broadcasted_iota(
            jnp.int32, (_NUM_BOXES, hgt), 1).astype(jnp.float32)
        xm = (((xx >= x1) & (xx < x2)).astype(jnp.float32)
              * valid).astype(jnp.bfloat16)
        ym = ((yy >= y1) & (yy < y2)).astype(jnp.bfloat16)
        cov = jax.lax.dot_general(
            ym, xm, (((0,), (0,)), ((), ())),
            preferred_element_type=jnp.float32,
        )
        areas.append(jnp.sum((cov > 0.5).astype(jnp.float32)))

    total = jnp.float32(0.0)
    for cp, v_ref, area, (n, hgt, wid) in zip(
            (cp0, cp1, cp2), (v0_ref, v1_ref, v2_ref), areas, _LEVELS):
        cp.wait()
        s = jnp.sum(v_ref[:, :])
        li = (s / float(n * hgt * wid) - area / float(hgt * wid)) ** 2
        total = total + li

    out_ref[:, :] = jnp.reshape(total / 3.0, (1, 1))


def kernel(h0, h1, h2, label, im_dimx, im_dimy):
    h0f = h0.reshape(8 * 200, 336)
    h1f = h1.reshape(8 * 100, 168)
    h2f = h2.reshape(8 * 50, 84)
    dx = jnp.asarray(im_dimx, jnp.float32).reshape(1, 1)
    dy = jnp.asarray(im_dimy, jnp.float32).reshape(1, 1)
    out = pl.pallas_call(
        _lamm_body,
        in_specs=[
            pl.BlockSpec(memory_space=pl.ANY),
            pl.BlockSpec(memory_space=pl.ANY),
            pl.BlockSpec(memory_space=pl.ANY),
            pl.BlockSpec(memory_space=pltpu.MemorySpace.VMEM),
            pl.BlockSpec(memory_space=pltpu.MemorySpace.SMEM),
            pl.BlockSpec(memory_space=pltpu.MemorySpace.SMEM),
        ],
        out_shape=jax.ShapeDtypeStruct((1, 1), jnp.float32),
        scratch_shapes=[
            pltpu.VMEM((8 * 200, 336), jnp.float32),
            pltpu.VMEM((8 * 100, 168), jnp.float32),
            pltpu.VMEM((8 * 50, 84), jnp.float32),
            pltpu.SemaphoreType.DMA,
            pltpu.SemaphoreType.DMA,
            pltpu.SemaphoreType.DMA,
        ],
    )(h0f, h1f, h2f, label, dx, dy)
    return out.reshape(())


# speedup vs baseline: 1.1427x; 1.1427x over previous
import jax
import jax.numpy as jnp
from jax.experimental import pallas as pl
from jax.experimental.pallas import tpu as pltpu

_NUM_BOXES = 100
_LEVELS = ((8, 200, 336), (8, 100, 168), (8, 50, 84))


def _lamm_body(h0_ref, h1_ref, h2_ref, lab_ref, dx_ref, dy_ref, out_ref):
    dimx = dx_ref[0, 0]
    dimy = dy_ref[0, 0]
    lab = lab_ref[:, :]

    areas = []
    for _, hgt, wid in _LEVELS:
        sx = wid / dimx
        sy = hgt / dimy
        x1 = jnp.clip(jnp.round(lab[:, 0:1] * sx), 0.0, wid - 1.0)
        y1 = jnp.clip(jnp.round(lab[:, 1:2] * sy), 0.0, hgt - 1.0)
        x2 = jnp.clip(jnp.round(lab[:, 2:3] * sx), 0.0, float(wid))
        y2 = jnp.clip(jnp.round(lab[:, 3:4] * sy), 0.0, float(hgt))
        valid = ((x2 > x1) & (y2 > y1)).astype(jnp.float32)
        xx = jax.lax.broadcasted_iota(
            jnp.int32, (_NUM_BOXES, wid), 1).astype(jnp.float32)
        yy = jax.lax.broadcasted_iota(
            jnp.int32, (_NUM_BOXES, hgt), 1).astype(jnp.float32)
        xm = (((xx >= x1) & (xx < x2)).astype(jnp.float32)
              * valid).astype(jnp.bfloat16)
        ym = ((yy >= y1) & (yy < y2)).astype(jnp.bfloat16)
        cov = jax.lax.dot_general(
            ym, xm, (((0,), (0,)), ((), ())),
            preferred_element_type=jnp.float32,
        )
        areas.append(jnp.sum((cov > 0.5).astype(jnp.float32)))

    total = jnp.float32(0.0)
    for h_ref, area, (n, hgt, wid) in zip(
            (h0_ref, h1_ref, h2_ref), areas, _LEVELS):
        s = jnp.sum(h_ref[:, :])
        li = (s / float(n * hgt * wid) - area / float(hgt * wid)) ** 2
        total = total + li

    out_ref[:, :] = jnp.reshape(total / 3.0, (1, 1))


def kernel(h0, h1, h2, label, im_dimx, im_dimy):
    h0f = h0.reshape(8 * 200, 336)
    h1f = h1.reshape(8 * 100, 168)
    h2f = h2.reshape(8 * 50, 84)
    dx = jnp.asarray(im_dimx, jnp.float32).reshape(1, 1)
    dy = jnp.asarray(im_dimy, jnp.float32).reshape(1, 1)
    out = pl.pallas_call(
        _lamm_body,
        in_specs=[
            pl.BlockSpec(memory_space=pltpu.MemorySpace.VMEM),
            pl.BlockSpec(memory_space=pltpu.MemorySpace.VMEM),
            pl.BlockSpec(memory_space=pltpu.MemorySpace.VMEM),
            pl.BlockSpec(memory_space=pltpu.MemorySpace.VMEM),
            pl.BlockSpec(memory_space=pltpu.MemorySpace.SMEM),
            pl.BlockSpec(memory_space=pltpu.MemorySpace.SMEM),
        ],
        out_shape=jax.ShapeDtypeStruct((1, 1), jnp.float32),
    )(h0f, h1f, h2f, label, dx, dy)
    return out.reshape(())
